# Initial kernel scaffold; baseline (speedup 1.0000x reference)
import jax
import jax.numpy as jnp
from jax.experimental import pallas as pl


def kernel(x0, x1, x2, x3, x4, x5, x6, x7):
    raise NotImplementedError("write your pallas kernel here")


# speedup vs baseline: 20.3963x; 20.3963x over previous
import functools

import jax
import jax.numpy as jnp
from jax import lax
from jax.experimental import pallas as pl
from jax.experimental.pallas import tpu as pltpu
from jax.experimental.pallas import tpu_sc as plsc

N = 1_000_000
NPROJ = 8
MIN_RATIO = 1500.0 / 30000000.0
L = 16
HBINS = 65536


def _iota():
    return lax.broadcasted_iota(jnp.int32, (L,), 0)


def _lane(v, g):
    return jnp.sum(jnp.where(_iota() == g, v, 0))


def _zero_hist(h):
    z = jnp.zeros((L,), jnp.int32)

    def body(i, _):
        b = i * (L * 8)
        for u in range(8):
            h[pl.ds(b + u * L, L)] = z
        return 0

    lax.fori_loop(0, HBINS // (L * 8), body, 0)


def _child_totals(h, base, span):
    child = span // 16
    if child == 1:
        return h[pl.ds(base, L)]
    nv = child // L
    T = jnp.zeros((L,), jnp.int32)
    for g in range(16):
        start = base + g * child
        if nv <= 4:
            acc = jnp.zeros((L,), jnp.int32)
            for j in range(nv):
                acc = acc + h[pl.ds(start + j * L, L)]
        else:
            def body(j, acc, start=start):
                b = start + j * (L * 4)
                for u in range(4):
                    acc = acc + h[pl.ds(b + u * L, L)]
                return acc

            acc = lax.fori_loop(0, nv // 4, body, jnp.zeros((L,), jnp.int32))
        T = jnp.where(_iota() == g, jnp.sum(acc), T)
    return T


def _find(h, r, T0):
    base_bin = jnp.int32(0)
    cbelow = jnp.int32(0)
    span = HBINS
    T = T0
    while True:
        child = span // 16
        inc = plsc.cumsum(T)
        g = jnp.max(plsc.all_reduce_ffs((cbelow + inc) > r))
        cbelow = cbelow + _lane(inc, g) - _lane(T, g)
        base_bin = base_bin + g * child
        if child == 1:
            return base_bin, cbelow
        span = child
        T = _child_totals(h, base_bin, span)


def _build_sc_median(n, chunk, unroll, interpret=False):
    nchunk = n // chunk
    niter = chunk // L // unroll
    assert nchunk * chunk == n and niter * unroll * L == chunk
    r0 = n - (n // 2 + 1)
    r1 = n // 2

    mesh = plsc.VectorSubcoreMesh(
        core_axis_name="c", subcore_axis_name="s", num_cores=2, num_subcores=16
    )

    @functools.partial(
        pl.kernel,
        out_type=jax.ShapeDtypeStruct((NPROJ, L), jnp.float32),
        mesh=mesh,
        interpret=interpret,
        compiler_params=pltpu.CompilerParams(needs_layout_passes=False),
        scratch_types=[
            pltpu.VMEM((chunk,), jnp.float32),
            pltpu.VMEM((HBINS,), jnp.int32),
            pltpu.VMEM((L,), jnp.float32),
        ],
    )
    def sc_median(x0, x1, x2, x3, x4, x5, x6, x7, out, buf, hist, med_v):
        xs = [x0, x1, x2, x3, x4, x5, x6, x7]
        c = lax.axis_index("c")
        s = lax.axis_index("s")
        p = c * 4 + s

        @pl.when(s < 4)
        def _():
            ones = jnp.ones((L,), jnp.int32)

            def load_chunk(ci):
                off = pl.multiple_of(ci * chunk, 8)
                for i in range(NPROJ):
                    @pl.when(p == i)
                    def _(i=i):
                        pltpu.sync_copy(xs[i].at[pl.ds(off, chunk)], buf)

            _zero_hist(hist)

            def chunk1(ci, _):
                load_chunk(ci)

                def body(j, _):
                    b = j * (L * unroll)
                    for u in range(unroll):
                        v = buf[pl.ds(b + u * L, L)]
                        bits = lax.bitcast_convert_type(v, jnp.int32)
                        hi = lax.shift_right_logical(bits, 16)
                        plsc.addupdate_scatter(hist, [hi], ones)
                    return 0

                lax.fori_loop(0, niter, body, 0)
                return 0

            lax.fori_loop(0, nchunk, chunk1, 0)

            T0 = _child_totals(hist, 0, HBINS)
            b_a, c_a = _find(hist, jnp.int32(r0), T0)
            b_b, _ = _find(hist, jnp.int32(r1), T0)

            _zero_hist(hist)

            def chunk2(ci, runmin):
                load_chunk(ci)

                def body(j, rm):
                    b = j * (L * unroll)
                    for u in range(unroll):
                        v = buf[pl.ds(b + u * L, L)]
                        bits = lax.bitcast_convert_type(v, jnp.int32)
                        hi = lax.shift_right_logical(bits, 16)
                        lo = lax.bitwise_and(bits, 0xFFFF)
                        plsc.addupdate_scatter(hist, [lo], ones, mask=hi == b_a)
                        rm = jnp.minimum(
                            rm, jnp.where(hi == b_b, lo, jnp.int32(0x10000))
                        )
                    return rm

                return lax.fori_loop(0, niter, body, runmin)

            runmin = lax.fori_loop(
                0, nchunk, chunk2, jnp.full((L,), 0x10000, jnp.int32)
            )

            T02 = _child_totals(hist, 0, HBINS)
            n_a = jnp.sum(T02)
            lo_a, _ = _find(hist, jnp.int32(r0) - c_a, T02)
            r1p = jnp.minimum(jnp.int32(r1) - c_a, n_a - 1)
            lo_b, _ = _find(hist, r1p, T02)
            minlow = jnp.min(runmin)

            same = b_a == b_b
            v0_bits = lax.shift_left(b_a, 16) | lo_a
            v1_bits = jnp.where(
                same,
                lax.shift_left(b_a, 16) | lo_b,
                lax.shift_left(b_b, 16) | minlow,
            )
            f0 = lax.bitcast_convert_type(jnp.full((L,), v0_bits), jnp.float32)
            f1 = lax.bitcast_convert_type(jnp.full((L,), v1_bits), jnp.float32)
            med_v[...] = (f0 + f1) * 0.5
            pltpu.sync_copy(med_v, out.at[p])

    return sc_median


_sc_median = _build_sc_median(N, 50000, 25)


def _combine_body(m_ref, o_ref):
    med = m_ref[...][:, 0:1]
    total = jnp.sum(med)
    ratio = med / total
    meets = (ratio >= jnp.float32(MIN_RATIO)).astype(jnp.float32)
    o_ref[...] = 30000000 * ratio * meets


@jax.jit
def kernel(x0, x1, x2, x3, x4, x5, x6, x7):
    meds = _sc_median(x0, x1, x2, x3, x4, x5, x6, x7)
    return pl.pallas_call(
        _combine_body,
        out_shape=jax.ShapeDtypeStruct((NPROJ, 1), jnp.float32),
    )(meds)


# speedup vs baseline: 69.9403x; 3.4291x over previous
import functools

import jax
import jax.numpy as jnp
from jax import lax
from jax.experimental import pallas as pl
from jax.experimental.pallas import tpu as pltpu
from jax.experimental.pallas import tpu_sc as plsc

N = 1_000_000
NPROJ = 8
MIN_RATIO = 1500.0 / 30000000.0
L = 16
HBINS = 65536
TPP = 4


def _iota():
    return lax.broadcasted_iota(jnp.int32, (L,), 0)


def _lane(v, g):
    return jnp.sum(jnp.where(_iota() == g, v, 0))


def _zero_hist(h):
    z = jnp.zeros((L,), jnp.int32)

    def body(i, _):
        b = i * (L * 8)
        for u in range(8):
            h[pl.ds(b + u * L, L)] = z
        return 0

    lax.fori_loop(0, HBINS // (L * 8), body, 0)


def _child_totals(h, base, span):
    child = span // 16
    if child == 1:
        return h[pl.ds(base, L)]
    nv = child // L
    T = jnp.zeros((L,), jnp.int32)
    for g in range(16):
        start = base + g * child
        if nv <= 4:
            acc = jnp.zeros((L,), jnp.int32)
            for j in range(nv):
                acc = acc + h[pl.ds(start + j * L, L)]
        else:
            def body(j, acc, start=start):
                b = start + j * (L * 4)
                for u in range(4):
                    acc = acc + h[pl.ds(b + u * L, L)]
                return acc

            acc = lax.fori_loop(0, nv // 4, body, jnp.zeros((L,), jnp.int32))
        T = jnp.where(_iota() == g, jnp.sum(acc), T)
    return T


def _find(h, r, T0, exch):
    base_bin = jnp.int32(0)
    cbelow = jnp.int32(0)
    span = HBINS
    T = T0
    while True:
        child = span // 16
        inc = plsc.cumsum(T)
        g = jnp.max(plsc.all_reduce_ffs((cbelow + inc) > r))
        cbelow = cbelow + _lane(inc, g) - _lane(T, g)
        base_bin = base_bin + g * child
        if child == 1:
            return base_bin, cbelow
        span = child
        T = exch(_child_totals(h, base_bin, span))


def _build_sc_median(n, chunk, unroll, interpret=False):
    per_tile = n // TPP
    nchunk = per_tile // chunk
    niter = chunk // L // unroll
    assert nchunk * chunk == per_tile and niter * unroll * L == chunk
    r0 = n - (n // 2 + 1)
    r1 = n // 2

    mesh = plsc.VectorSubcoreMesh(
        core_axis_name="c", subcore_axis_name="s", num_cores=2, num_subcores=16
    )

    @functools.partial(
        pl.kernel,
        out_type=jax.ShapeDtypeStruct((NPROJ, L), jnp.float32),
        mesh=mesh,
        interpret=interpret,
        compiler_params=pltpu.CompilerParams(needs_layout_passes=False),
        scratch_types=[
            pltpu.VMEM((chunk,), jnp.float32),
            pltpu.VMEM((HBINS,), jnp.int32),
            pltpu.VMEM((L,), jnp.float32),
            pltpu.VMEM((L,), jnp.int32),
            pltpu.VMEM((L,), jnp.int32),
            pltpu.VMEM_SHARED((32, L), jnp.int32),
        ],
    )
    def sc_median(x0, x1, x2, x3, x4, x5, x6, x7, out, buf, hist, med_v,
                  xbuf, tbuf, shr):
        xs = [x0, x1, x2, x3, x4, x5, x6, x7]
        c = lax.axis_index("c")
        s = lax.axis_index("s")
        p_l = lax.div(s, TPP)
        q = lax.rem(s, TPP)
        p = c * 4 + p_l
        ones = jnp.ones((L,), jnp.int32)

        def exchange(vec, combine):
            xbuf[...] = vec
            pltpu.sync_copy(xbuf, shr.at[16 + s])
            plsc.subcore_barrier()
            acc = None
            for r in range(TPP):
                pltpu.sync_copy(shr.at[16 + p_l * TPP + r], tbuf)
                t = tbuf[...]
                acc = t if acc is None else combine(acc, t)
            plsc.subcore_barrier()
            return acc

        exch_sum = lambda v: exchange(v, jnp.add)
        exch_min = lambda v: exchange(v, jnp.minimum)

        def load_chunk(ci):
            off = pl.multiple_of(q * per_tile + ci * chunk, 8)
            for i in range(NPROJ):
                @pl.when(p == i)
                def _(i=i):
                    pltpu.sync_copy(xs[i].at[pl.ds(off, chunk)], buf)

        _zero_hist(hist)

        def chunk1(ci, _):
            load_chunk(ci)

            def body(j, _):
                b = j * (L * unroll)
                for u in range(unroll):
                    v = buf[pl.ds(b + u * L, L)]
                    bits = lax.bitcast_convert_type(v, jnp.int32)
                    hi = lax.shift_right_logical(bits, 16)
                    plsc.addupdate_scatter(hist, [hi], ones)
                return 0

            lax.fori_loop(0, niter, body, 0)
            return 0

        lax.fori_loop(0, nchunk, chunk1, 0)

        T0 = exch_sum(_child_totals(hist, 0, HBINS))
        b_a, c_a = _find(hist, jnp.int32(r0), T0, exch_sum)
        b_b, _ = _find(hist, jnp.int32(r1), T0, exch_sum)

        _zero_hist(hist)

        def chunk2(ci, runmin):
            load_chunk(ci)

            def body(j, rm):
                b = j * (L * unroll)
                for u in range(unroll):
                    v = buf[pl.ds(b + u * L, L)]
                    bits = lax.bitcast_convert_type(v, jnp.int32)
                    hi = lax.shift_right_logical(bits, 16)
                    lo = lax.bitwise_and(bits, 0xFFFF)
                    plsc.addupdate_scatter(hist, [lo], ones, mask=hi == b_a)
                    rm = jnp.minimum(
                        rm, jnp.where(hi == b_b, lo, jnp.int32(0x10000))
                    )
                return rm

            return lax.fori_loop(0, niter, body, runmin)

        runmin = lax.fori_loop(
            0, nchunk, chunk2, jnp.full((L,), 0x10000, jnp.int32)
        )
        runmin = exch_min(runmin)

        T02 = exch_sum(_child_totals(hist, 0, HBINS))
        n_a = jnp.sum(T02)
        lo_a, _ = _find(hist, jnp.int32(r0) - c_a, T02, exch_sum)
        r1p = jnp.minimum(jnp.int32(r1) - c_a, n_a - 1)
        lo_b, _ = _find(hist, r1p, T02, exch_sum)
        minlow = jnp.min(runmin)

        same = b_a == b_b
        v0_bits = lax.shift_left(b_a, 16) | lo_a
        v1_bits = jnp.where(
            same,
            lax.shift_left(b_a, 16) | lo_b,
            lax.shift_left(b_b, 16) | minlow,
        )
        f0 = lax.bitcast_convert_type(jnp.full((L,), v0_bits), jnp.float32)
        f1 = lax.bitcast_convert_type(jnp.full((L,), v1_bits), jnp.float32)
        med_v[...] = (f0 + f1) * 0.5

        @pl.when(q == 0)
        def _():
            pltpu.sync_copy(med_v, out.at[p])

    return sc_median


_sc_median = _build_sc_median(N, 50000, 25)


def _combine_body(m_ref, o_ref):
    med = m_ref[...][:, 0:1]
    total = jnp.sum(med)
    ratio = med / total
    meets = (ratio >= jnp.float32(MIN_RATIO)).astype(jnp.float32)
    o_ref[...] = 30000000 * ratio * meets


@jax.jit
def kernel(x0, x1, x2, x3, x4, x5, x6, x7):
    meds = _sc_median(x0, x1, x2, x3, x4, x5, x6, x7)
    return pl.pallas_call(
        _combine_body,
        out_shape=jax.ShapeDtypeStruct((NPROJ, 1), jnp.float32),
    )(meds)


# speedup vs baseline: 206.3226x; 2.9500x over previous
import functools

import jax
import jax.numpy as jnp
from jax import lax
from jax.experimental import pallas as pl
from jax.experimental.pallas import tpu as pltpu
from jax.experimental.pallas import tpu_sc as plsc

N = 1_000_000
NPROJ = 8
MIN_RATIO = 1500.0 / 30000000.0
L = 16
HBINS = 65536
TPP = 4


def _iota():
    return lax.broadcasted_iota(jnp.int32, (L,), 0)


def _lane(v, g):
    return jnp.sum(jnp.where(_iota() == g, v, 0))


def _zero_hist(h):
    z = jnp.zeros((L,), jnp.int32)

    @plsc.parallel_loop(0, HBINS // L, unroll=8)
    def _(i):
        h[pl.ds(i * L, L)] = z


def _child_totals(h, base, span):
    child = span // 16
    if child == 1:
        return h[pl.ds(base, L)]
    nv = child // L
    T = jnp.zeros((L,), jnp.int32)
    for g in range(16):
        start = base + g * child
        if nv <= 4:
            acc = jnp.zeros((L,), jnp.int32)
            for j in range(nv):
                acc = acc + h[pl.ds(start + j * L, L)]
        else:
            @plsc.parallel_loop(0, nv, unroll=8,
                                carry=jnp.zeros((L,), jnp.int32))
            def acc(j, a, start=start):
                return a + h[pl.ds(start + j * L, L)]
        T = jnp.where(_iota() == g, jnp.sum(acc), T)
    return T


def _find(h, r, T0, exch):
    base_bin = jnp.int32(0)
    cbelow = jnp.int32(0)
    span = HBINS
    T = T0
    while True:
        child = span // 16
        inc = plsc.cumsum(T)
        g = jnp.max(plsc.all_reduce_ffs((cbelow + inc) > r))
        cbelow = cbelow + _lane(inc, g) - _lane(T, g)
        base_bin = base_bin + g * child
        if child == 1:
            return base_bin, cbelow
        span = child
        T = exch(_child_totals(h, base_bin, span))


def _build_sc_median(n, chunk, unroll, interpret=False):
    per_tile = n // TPP
    nchunk = per_tile // chunk
    vpc = chunk // L
    assert nchunk * chunk == per_tile and vpc * L == chunk
    assert nchunk % 2 == 1 and vpc % unroll == 0
    npair = (nchunk - 1) // 2
    r0 = n - (n // 2 + 1)
    r1 = n // 2

    mesh = plsc.VectorSubcoreMesh(
        core_axis_name="c", subcore_axis_name="s", num_cores=2, num_subcores=16
    )

    @functools.partial(
        pl.kernel,
        out_type=jax.ShapeDtypeStruct((NPROJ, L), jnp.float32),
        mesh=mesh,
        interpret=interpret,
        compiler_params=pltpu.CompilerParams(needs_layout_passes=False),
        scratch_types=[
            pltpu.VMEM((chunk,), jnp.float32),
            pltpu.VMEM((chunk,), jnp.float32),
            pltpu.VMEM((HBINS,), jnp.int32),
            pltpu.VMEM((L,), jnp.float32),
            pltpu.VMEM((L,), jnp.int32),
            pltpu.VMEM((L,), jnp.int32),
            pltpu.VMEM_SHARED((32, L), jnp.int32),
            pltpu.SemaphoreType.DMA,
            pltpu.SemaphoreType.DMA,
        ],
    )
    def sc_median(x0, x1, x2, x3, x4, x5, x6, x7, out, buf0, buf1, hist,
                  med_v, xbuf, tbuf, shr, sem0, sem1):
        xs = [x0, x1, x2, x3, x4, x5, x6, x7]
        c = lax.axis_index("c")
        s = lax.axis_index("s")
        p_l = lax.div(s, TPP)
        q = lax.rem(s, TPP)
        p = c * 4 + p_l
        ones = jnp.ones((L,), jnp.int32)

        def exchange(vec, combine):
            xbuf[...] = vec
            pltpu.sync_copy(xbuf, shr.at[16 + s])
            plsc.subcore_barrier()
            acc = None
            for r in range(TPP):
                pltpu.sync_copy(shr.at[16 + p_l * TPP + r], tbuf)
                t = tbuf[...]
                acc = t if acc is None else combine(acc, t)
            plsc.subcore_barrier()
            return acc

        exch_sum = lambda v: exchange(v, jnp.add)
        exch_min = lambda v: exchange(v, jnp.minimum)

        def start_load(ci, buf, sem):
            off = pl.multiple_of(q * per_tile + ci * chunk, 8)
            for i in range(NPROJ):
                @pl.when(p == i)
                def _(i=i):
                    pltpu.make_async_copy(
                        xs[i].at[pl.ds(off, chunk)], buf, sem
                    ).start()

        def wait_load(buf, sem):
            pltpu.make_async_copy(x0.at[pl.ds(0, chunk)], buf, sem).wait()

        def stream(process, carry):
            start_load(0, buf0, sem0)

            def pair(g, carry):
                a = 2 * g
                wait_load(buf0, sem0)
                start_load(a + 1, buf1, sem1)
                carry = process(buf0, carry)
                wait_load(buf1, sem1)
                start_load(a + 2, buf0, sem0)
                return process(buf1, carry)

            carry = lax.fori_loop(0, npair, pair, carry)
            wait_load(buf0, sem0)
            return process(buf0, carry)

        _zero_hist(hist)

        def proc1(buf, carry):
            @plsc.parallel_loop(0, vpc, unroll=unroll)
            def _(j):
                v = buf[pl.ds(j * L, L)]
                bits = lax.bitcast_convert_type(v, jnp.int32)
                hi = lax.shift_right_logical(bits, 16)
                plsc.addupdate_scatter(hist, [hi], ones)

            return carry

        stream(proc1, jnp.int32(0))

        T0 = exch_sum(_child_totals(hist, 0, HBINS))
        b_a, c_a = _find(hist, jnp.int32(r0), T0, exch_sum)
        b_b, _ = _find(hist, jnp.int32(r1), T0, exch_sum)

        _zero_hist(hist)

        def proc2(buf, runmin):
            @plsc.parallel_loop(0, vpc, unroll=unroll, carry=runmin)
            def runmin(j, rm):
                v = buf[pl.ds(j * L, L)]
                bits = lax.bitcast_convert_type(v, jnp.int32)
                hi = lax.shift_right_logical(bits, 16)
                lo = lax.bitwise_and(bits, 0xFFFF)
                plsc.addupdate_scatter(hist, [lo], ones, mask=hi == b_a)
                return jnp.minimum(
                    rm, jnp.where(hi == b_b, lo, jnp.int32(0x10000))
                )

            return runmin

        runmin = stream(proc2, jnp.full((L,), 0x10000, jnp.int32))
        runmin = exch_min(runmin)

        T02 = exch_sum(_child_totals(hist, 0, HBINS))
        n_a = jnp.sum(T02)
        lo_a, _ = _find(hist, jnp.int32(r0) - c_a, T02, exch_sum)
        r1p = jnp.minimum(jnp.int32(r1) - c_a, n_a - 1)
        lo_b, _ = _find(hist, r1p, T02, exch_sum)
        minlow = jnp.min(runmin)

        same = b_a == b_b
        v0_bits = lax.shift_left(b_a, 16) | lo_a
        v1_bits = jnp.where(
            same,
            lax.shift_left(b_a, 16) | lo_b,
            lax.shift_left(b_b, 16) | minlow,
        )
        f0 = lax.bitcast_convert_type(jnp.full((L,), v0_bits), jnp.float32)
        f1 = lax.bitcast_convert_type(jnp.full((L,), v1_bits), jnp.float32)
        med_v[...] = (f0 + f1) * 0.5

        @pl.when(q == 0)
        def _():
            pltpu.sync_copy(med_v, out.at[p])

    return sc_median


_sc_median = _build_sc_median(N, 10000, 5)


def _combine_body(m_ref, o_ref):
    med = m_ref[...][:, 0:1]
    total = jnp.sum(med)
    ratio = med / total
    meets = (ratio >= jnp.float32(MIN_RATIO)).astype(jnp.float32)
    o_ref[...] = 30000000 * ratio * meets


@jax.jit
def kernel(x0, x1, x2, x3, x4, x5, x6, x7):
    meds = _sc_median(x0, x1, x2, x3, x4, x5, x6, x7)
    return pl.pallas_call(
        _combine_body,
        out_shape=jax.ShapeDtypeStruct((NPROJ, 1), jnp.float32),
    )(meds)
